# Initial kernel scaffold; baseline (speedup 1.0000x reference)
import jax
import jax.numpy as jnp
from jax.experimental import pallas as pl


def kernel(x, edge_index, edge_attr, batch, node_emb, W_e, b_e, eps, W1, b1, bn1_gamma, bn1_beta, W2, b2, bn_gamma, bn_beta):
    raise NotImplementedError("write your pallas kernel here")


# speedup vs baseline: 2.0549x; 2.0549x over previous
import functools

import jax
import jax.numpy as jnp
from jax import lax
from jax.experimental import pallas as pl
from jax.experimental.pallas import tpu as pltpu
from jax.experimental.pallas import tpu_sc as plsc

N = 10000
E = 320000
D = 128
H = D // 2
L = 3
EDGE_DIM = 7

NC = 2
NS = 16
CHUNK = 80
EPT = E // NS
NCHUNK = EPT // CHUNK
N_PAD = 10240
ZROWS = 128
RPT = N_PAD // NS

_HIGHEST = jax.lax.Precision.HIGHEST


def _dot(a, b):
    return lax.dot_general(a, b, (((1,), (0,)), ((), ())),
                           preferred_element_type=jnp.float32)


def _edge_mlp_body(ea_ref, w_ref, brow_ref, out_ref, *, relu):
    z = _dot(ea_ref[...], w_ref[0]) + brow_ref[0]
    if relu:
        z = jnp.maximum(z, 0.0)
    out_ref[0] = z


def _edge_mlp(edge_attr, W, brow, relu):
    W2c = W.reshape(EDGE_DIM, NC, H).transpose(1, 0, 2)
    b2c = brow.reshape(1, NC, H).transpose(1, 0, 2)
    BE = 4000
    return pl.pallas_call(
        functools.partial(_edge_mlp_body, relu=relu),
        grid=(NC, E // BE),
        in_specs=[
            pl.BlockSpec((BE, EDGE_DIM), lambda h, i: (i, 0)),
            pl.BlockSpec((1, EDGE_DIM, H), lambda h, i: (h, 0, 0)),
            pl.BlockSpec((1, 1, H), lambda h, i: (h, 0, 0)),
        ],
        out_specs=pl.BlockSpec((1, BE, H), lambda h, i: (h, i, 0)),
        out_shape=jax.ShapeDtypeStruct((NC, E, H), jnp.float32),
    )(edge_attr, W2c, b2c)


def _dense_body(h_ref, p_ref, sc_ref, W1_ref, b1_ref, g1_ref, be1_ref,
                W2_ref, b2_ref, g2_ref, be2_ref, out_ref, *, h_is_row, split_out):
    agg = jnp.concatenate([p_ref[0, :N], p_ref[1, :N]], axis=1)
    if h_is_row:
        h = h_ref[...]
    else:
        h = jnp.concatenate([h_ref[:N], h_ref[N:]], axis=1)
    z = sc_ref[0, 0] * h + agg
    z = _dot(z, W1_ref[...]) + b1_ref[...]
    mu = jnp.mean(z, axis=0, keepdims=True)
    zc = z - mu
    var = jnp.mean(zc * zc, axis=0, keepdims=True)
    z = g1_ref[...] * zc * lax.rsqrt(var + 1e-5) + be1_ref[...]
    z = jnp.maximum(z, 0.0)
    z = _dot(z, W2_ref[...]) + b2_ref[...]
    mu2 = jnp.mean(z, axis=0, keepdims=True)
    zc2 = z - mu2
    var2 = jnp.mean(zc2 * zc2, axis=0, keepdims=True)
    z = g2_ref[...] * zc2 * lax.rsqrt(var2 + 1e-5) + be2_ref[...]
    if split_out:
        z = jnp.maximum(z, 0.0)
        out_ref[:N] = z[:, :H]
        out_ref[N:] = z[:, H:]
    else:
        out_ref[...] = z


def _dense(h, p, scale, W1, b1, g1, be1, W2, b2, g2, be2, h_is_row, split_out):
    out_shape = (jax.ShapeDtypeStruct((2 * N, H), jnp.float32) if split_out
                 else jax.ShapeDtypeStruct((N, D), jnp.float32))
    return pl.pallas_call(
        functools.partial(_dense_body, h_is_row=h_is_row, split_out=split_out),
        out_shape=out_shape,
    )(h, p, scale, W1, b1, g1, be1, W2, b2, g2, be2)


def _sc_scatter_body(msg_hbm, dst_hbm, out_hbm, dst_v, msg_v, zbuf, agg_sh):
    cid = lax.axis_index("c")
    sid = lax.axis_index("s")

    @pl.loop(0, ZROWS)
    def _(r):
        for c in range(H // 16):
            zbuf[r, pl.ds(c * 16, 16)] = jnp.zeros((16,), jnp.float32)

    for k in range(RPT // ZROWS):
        pltpu.sync_copy(zbuf, agg_sh.at[pl.ds(sid * RPT + k * ZROWS, ZROWS)])
    plsc.subcore_barrier()

    pltpu.sync_copy(dst_hbm.at[sid], dst_v)

    @pl.loop(0, NCHUNK)
    def _(k):
        eb = (sid * NCHUNK + k) * CHUNK
        pltpu.sync_copy(msg_hbm.at[cid, pl.ds(eb, CHUNK)], msg_v)
        pltpu.sync_copy(msg_v, agg_sh.at[dst_v.at[k]], add=True)

    plsc.subcore_barrier()
    for k in range(RPT // ZROWS):
        r0 = sid * RPT + k * ZROWS
        pltpu.sync_copy(agg_sh.at[pl.ds(r0, ZROWS)], out_hbm.at[cid, pl.ds(r0, ZROWS)])


def _sc_scatter(msg, dst3d):
    mesh = plsc.VectorSubcoreMesh(core_axis_name="c", subcore_axis_name="s")
    f = pl.kernel(
        _sc_scatter_body,
        out_type=jax.ShapeDtypeStruct((NC, N_PAD, H), jnp.float32),
        mesh=mesh,
        scratch_types=[
            pltpu.VMEM((NCHUNK, CHUNK), jnp.int32),
            pltpu.VMEM((CHUNK, H), jnp.float32),
            pltpu.VMEM((ZROWS, H), jnp.float32),
            pltpu.VMEM_SHARED((N_PAD, H), jnp.float32),
        ],
        compiler_params=pltpu.CompilerParams(use_tc_tiling_on_sc=False),
    )
    return f(msg, dst3d)


def _sc_msg_body(h_hbm, ee_hbm, src_hbm, dst_hbm, out_hbm,
                 src_v, dst_v, rows_v, ee_v, zbuf, agg_sh):
    cid = lax.axis_index("c")
    sid = lax.axis_index("s")

    @pl.loop(0, ZROWS)
    def _(r):
        for c in range(H // 16):
            zbuf[r, pl.ds(c * 16, 16)] = jnp.zeros((16,), jnp.float32)

    for k in range(RPT // ZROWS):
        pltpu.sync_copy(zbuf, agg_sh.at[pl.ds(sid * RPT + k * ZROWS, ZROWS)])
    plsc.subcore_barrier()

    pltpu.sync_copy(src_hbm.at[sid], src_v)
    pltpu.sync_copy(dst_hbm.at[sid], dst_v)

    off = cid * N

    @pl.loop(0, NCHUNK)
    def _(r):
        for c in range(CHUNK // 16):
            s = pl.ds(c * 16, 16)
            src_v[r, s] = src_v[r, s] + off

    @pl.loop(0, NCHUNK)
    def _(k):
        eb = (sid * NCHUNK + k) * CHUNK
        pltpu.sync_copy(h_hbm.at[src_v.at[k]], rows_v)
        pltpu.sync_copy(ee_hbm.at[cid, pl.ds(eb, CHUNK)], ee_v)

        @pl.loop(0, CHUNK)
        def _(r):
            for c in range(H // 16):
                s = pl.ds(c * 16, 16)
                rows_v[r, s] = jnp.maximum(rows_v[r, s] + ee_v[r, s], 0.0)

        pltpu.sync_copy(rows_v, agg_sh.at[dst_v.at[k]], add=True)

    plsc.subcore_barrier()
    for k in range(RPT // ZROWS):
        r0 = sid * RPT + k * ZROWS
        pltpu.sync_copy(agg_sh.at[pl.ds(r0, ZROWS)], out_hbm.at[cid, pl.ds(r0, ZROWS)])


def _sc_msg(h_split, ee, src3d, dst3d):
    mesh = plsc.VectorSubcoreMesh(core_axis_name="c", subcore_axis_name="s")
    f = pl.kernel(
        _sc_msg_body,
        out_type=jax.ShapeDtypeStruct((NC, N_PAD, H), jnp.float32),
        mesh=mesh,
        scratch_types=[
            pltpu.VMEM((NCHUNK, CHUNK), jnp.int32),
            pltpu.VMEM((NCHUNK, CHUNK), jnp.int32),
            pltpu.VMEM((CHUNK, H), jnp.float32),
            pltpu.VMEM((CHUNK, H), jnp.float32),
            pltpu.VMEM((ZROWS, H), jnp.float32),
            pltpu.VMEM_SHARED((N_PAD, H), jnp.float32),
        ],
        compiler_params=pltpu.CompilerParams(use_tc_tiling_on_sc=False),
    )
    return f(h_split, ee, src3d, dst3d)


def kernel(x, edge_index, edge_attr, batch, node_emb, W_e, b_e, eps,
           W1, b1, bn1_gamma, bn1_beta, W2, b2, bn_gamma, bn_beta):
    src3d = edge_index[0].reshape(NS, NCHUNK, CHUNK)
    dst3d = edge_index[1].reshape(NS, NCHUNK, CHUNK)

    h0row = node_emb[0:1]

    scales = (1.0 + eps).reshape(L, 1, 1)

    msg0 = _edge_mlp(edge_attr, W_e[0], (b_e[0] + node_emb[0]).reshape(1, D), relu=True)
    p = _sc_scatter(msg0, dst3d)
    h = _dense(h0row, p, scales[0], W1[0], b1[0], bn1_gamma[0], bn1_beta[0],
               W2[0], b2[0], bn_gamma[0], bn_beta[0], h_is_row=True, split_out=True)

    for l in range(1, L):
        ee = _edge_mlp(edge_attr, W_e[l], b_e[l].reshape(1, D), relu=False)
        p = _sc_msg(h, ee, src3d, dst3d)
        h = _dense(h, p, scales[l], W1[l], b1[l], bn1_gamma[l], bn1_beta[l],
                   W2[l], b2[l], bn_gamma[l], bn_beta[l],
                   h_is_row=False, split_out=(l != L - 1))
    return h


# speedup vs baseline: 2.5215x; 1.2270x over previous
import functools

import jax
import jax.numpy as jnp
from jax import lax
from jax.experimental import pallas as pl
from jax.experimental.pallas import tpu as pltpu
from jax.experimental.pallas import tpu_sc as plsc

N = 10000
E = 320000
D = 128
H = D // 2
L = 3
EDGE_DIM = 7

NC = 2
NS = 16
CHUNK = 80
EPT = E // NS
NCHUNK = EPT // CHUNK
N_PAD = 10240
ZROWS = 128
RPT = N_PAD // NS

_HIGHEST = jax.lax.Precision.HIGHEST


def _dot(a, b):
    return lax.dot_general(a, b, (((1,), (0,)), ((), ())),
                           preferred_element_type=jnp.float32)


def _edge_mlp_body(ea_ref, w_ref, brow_ref, out_ref, *, relu):
    z = _dot(ea_ref[...], w_ref[0]) + brow_ref[0]
    if relu:
        z = jnp.maximum(z, 0.0)
    out_ref[0] = z


def _edge_mlp(edge_attr, W, brow, relu):
    W2c = W.reshape(EDGE_DIM, NC, H).transpose(1, 0, 2)
    b2c = brow.reshape(1, NC, H).transpose(1, 0, 2)
    BE = 4000
    return pl.pallas_call(
        functools.partial(_edge_mlp_body, relu=relu),
        grid=(NC, E // BE),
        in_specs=[
            pl.BlockSpec((BE, EDGE_DIM), lambda h, i: (i, 0)),
            pl.BlockSpec((1, EDGE_DIM, H), lambda h, i: (h, 0, 0)),
            pl.BlockSpec((1, 1, H), lambda h, i: (h, 0, 0)),
        ],
        out_specs=pl.BlockSpec((1, BE, H), lambda h, i: (h, i, 0)),
        out_shape=jax.ShapeDtypeStruct((NC, E, H), jnp.float32),
    )(edge_attr, W2c, b2c)


def _dense_body(h_ref, p_ref, sc_ref, W1_ref, b1_ref, g1_ref, be1_ref,
                W2_ref, b2_ref, g2_ref, be2_ref, out_ref, *, h_is_row, split_out):
    agg = jnp.concatenate([p_ref[0, :N], p_ref[1, :N]], axis=1)
    if h_is_row:
        h = h_ref[...]
    else:
        h = jnp.concatenate([h_ref[:N], h_ref[N:]], axis=1)
    z = sc_ref[0, 0] * h + agg
    z = _dot(z, W1_ref[...]) + b1_ref[...]
    mu = jnp.mean(z, axis=0, keepdims=True)
    zc = z - mu
    var = jnp.mean(zc * zc, axis=0, keepdims=True)
    z = g1_ref[...] * zc * lax.rsqrt(var + 1e-5) + be1_ref[...]
    z = jnp.maximum(z, 0.0)
    z = _dot(z, W2_ref[...]) + b2_ref[...]
    mu2 = jnp.mean(z, axis=0, keepdims=True)
    zc2 = z - mu2
    var2 = jnp.mean(zc2 * zc2, axis=0, keepdims=True)
    z = g2_ref[...] * zc2 * lax.rsqrt(var2 + 1e-5) + be2_ref[...]
    if split_out:
        z = jnp.maximum(z, 0.0)
        out_ref[:N] = z[:, :H]
        out_ref[N:] = z[:, H:]
    else:
        out_ref[...] = z


def _dense(h, p, scale, W1, b1, g1, be1, W2, b2, g2, be2, h_is_row, split_out):
    out_shape = (jax.ShapeDtypeStruct((2 * N, H), jnp.float32) if split_out
                 else jax.ShapeDtypeStruct((N, D), jnp.float32))
    return pl.pallas_call(
        functools.partial(_dense_body, h_is_row=h_is_row, split_out=split_out),
        out_shape=out_shape,
    )(h, p, scale, W1, b1, g1, be1, W2, b2, g2, be2)


def _sc_scatter_body(msg_hbm, dst_hbm, out_hbm, dst_v, msg_v, zbuf, agg_sh,
                     msem0, msem1, ssem0, ssem1):
    cid = lax.axis_index("c")
    sid = lax.axis_index("s")

    @pl.loop(0, ZROWS)
    def _(r):
        for c in range(H // 16):
            zbuf[r, pl.ds(c * 16, 16)] = jnp.zeros((16,), jnp.float32)

    for k in range(RPT // ZROWS):
        pltpu.sync_copy(zbuf, agg_sh.at[pl.ds(sid * RPT + k * ZROWS, ZROWS)])
    plsc.subcore_barrier()

    pltpu.sync_copy(dst_hbm.at[sid], dst_v)

    msg0, msg1 = msg_v.at[0], msg_v.at[1]

    def _load(k, buf, sem):
        eb = (sid * NCHUNK + k) * CHUNK
        return pltpu.make_async_copy(msg_hbm.at[cid, pl.ds(eb, CHUNK)], buf, sem)

    def _scat(k, buf, sem):
        return pltpu.make_async_copy(buf, agg_sh.at[dst_v.at[k]], sem)

    _load(0, msg0, msem0).start()

    @pl.loop(0, NCHUNK // 2)
    def _(j):
        k0 = 2 * j

        @pl.when(j > 0)
        def _():
            _scat(k0 - 1, msg1, ssem1).wait()

        _load(k0 + 1, msg1, msem1).start()
        _load(k0, msg0, msem0).wait()
        pltpu.async_copy(msg0, agg_sh.at[dst_v.at[k0]], ssem0, add=True)

        _load(k0 + 1, msg1, msem1).wait()
        pltpu.async_copy(msg1, agg_sh.at[dst_v.at[k0 + 1]], ssem1, add=True)

        _scat(k0, msg0, ssem0).wait()

        @pl.when(j < NCHUNK // 2 - 1)
        def _():
            _load(k0 + 2, msg0, msem0).start()

    _scat(NCHUNK - 1, msg1, ssem1).wait()
    plsc.subcore_barrier()
    for k in range(RPT // ZROWS):
        r0 = sid * RPT + k * ZROWS
        pltpu.sync_copy(agg_sh.at[pl.ds(r0, ZROWS)], out_hbm.at[cid, pl.ds(r0, ZROWS)])


def _sc_scatter(msg, dst3d):
    mesh = plsc.VectorSubcoreMesh(core_axis_name="c", subcore_axis_name="s")
    f = pl.kernel(
        _sc_scatter_body,
        out_type=jax.ShapeDtypeStruct((NC, N_PAD, H), jnp.float32),
        mesh=mesh,
        scratch_types=[
            pltpu.VMEM((NCHUNK, CHUNK), jnp.int32),
            pltpu.VMEM((2, CHUNK, H), jnp.float32),
            pltpu.VMEM((ZROWS, H), jnp.float32),
            pltpu.VMEM_SHARED((N_PAD, H), jnp.float32),
            pltpu.SemaphoreType.DMA,
            pltpu.SemaphoreType.DMA,
            pltpu.SemaphoreType.DMA,
            pltpu.SemaphoreType.DMA,
        ],
        compiler_params=pltpu.CompilerParams(use_tc_tiling_on_sc=False),
    )
    return f(msg, dst3d)


def _sc_msg_body(h_hbm, ee_hbm, src_hbm, dst_hbm, out_hbm,
                 src_v, dst_v, rows_v, ee_v, zbuf, agg_sh,
                 gsem0, gsem1, esem0, esem1, ssem0, ssem1):
    cid = lax.axis_index("c")
    sid = lax.axis_index("s")

    @pl.loop(0, ZROWS)
    def _(r):
        for c in range(H // 16):
            zbuf[r, pl.ds(c * 16, 16)] = jnp.zeros((16,), jnp.float32)

    for k in range(RPT // ZROWS):
        pltpu.sync_copy(zbuf, agg_sh.at[pl.ds(sid * RPT + k * ZROWS, ZROWS)])
    plsc.subcore_barrier()

    pltpu.sync_copy(src_hbm.at[sid], src_v)
    pltpu.sync_copy(dst_hbm.at[sid], dst_v)

    off = cid * N

    @pl.loop(0, NCHUNK)
    def _(r):
        for c in range(CHUNK // 16):
            s = pl.ds(c * 16, 16)
            src_v[r, s] = src_v[r, s] + off

    rows0, rows1 = rows_v.at[0], rows_v.at[1]
    ee0, ee1 = ee_v.at[0], ee_v.at[1]

    def _gat(k, buf, sem):
        return pltpu.make_async_copy(h_hbm.at[src_v.at[k]], buf, sem)

    def _lee(k, buf, sem):
        eb = (sid * NCHUNK + k) * CHUNK
        return pltpu.make_async_copy(ee_hbm.at[cid, pl.ds(eb, CHUNK)], buf, sem)

    def _scat(k, buf, sem):
        return pltpu.make_async_copy(buf, agg_sh.at[dst_v.at[k]], sem)

    def _compute(rows, eev):
        @pl.loop(0, CHUNK)
        def _(r):
            for c in range(H // 16):
                s = pl.ds(c * 16, 16)
                rows[r, s] = jnp.maximum(rows[r, s] + eev[r, s], 0.0)

    _gat(0, rows0, gsem0).start()
    _lee(0, ee0, esem0).start()

    @pl.loop(0, NCHUNK // 2)
    def _(j):
        k0 = 2 * j

        @pl.when(j > 0)
        def _():
            _scat(k0 - 1, rows1, ssem1).wait()

        _gat(k0 + 1, rows1, gsem1).start()
        _lee(k0 + 1, ee1, esem1).start()

        _gat(k0, rows0, gsem0).wait()
        _lee(k0, ee0, esem0).wait()
        _compute(rows0, ee0)
        pltpu.async_copy(rows0, agg_sh.at[dst_v.at[k0]], ssem0, add=True)

        _gat(k0 + 1, rows1, gsem1).wait()
        _lee(k0 + 1, ee1, esem1).wait()
        _compute(rows1, ee1)
        pltpu.async_copy(rows1, agg_sh.at[dst_v.at[k0 + 1]], ssem1, add=True)

        _scat(k0, rows0, ssem0).wait()

        @pl.when(j < NCHUNK // 2 - 1)
        def _():
            _gat(k0 + 2, rows0, gsem0).start()
            _lee(k0 + 2, ee0, esem0).start()

    _scat(NCHUNK - 1, rows1, ssem1).wait()
    plsc.subcore_barrier()
    for k in range(RPT // ZROWS):
        r0 = sid * RPT + k * ZROWS
        pltpu.sync_copy(agg_sh.at[pl.ds(r0, ZROWS)], out_hbm.at[cid, pl.ds(r0, ZROWS)])


def _sc_msg(h_split, ee, src3d, dst3d):
    mesh = plsc.VectorSubcoreMesh(core_axis_name="c", subcore_axis_name="s")
    f = pl.kernel(
        _sc_msg_body,
        out_type=jax.ShapeDtypeStruct((NC, N_PAD, H), jnp.float32),
        mesh=mesh,
        scratch_types=[
            pltpu.VMEM((NCHUNK, CHUNK), jnp.int32),
            pltpu.VMEM((NCHUNK, CHUNK), jnp.int32),
            pltpu.VMEM((2, CHUNK, H), jnp.float32),
            pltpu.VMEM((2, CHUNK, H), jnp.float32),
            pltpu.VMEM((ZROWS, H), jnp.float32),
            pltpu.VMEM_SHARED((N_PAD, H), jnp.float32),
            pltpu.SemaphoreType.DMA,
            pltpu.SemaphoreType.DMA,
            pltpu.SemaphoreType.DMA,
            pltpu.SemaphoreType.DMA,
            pltpu.SemaphoreType.DMA,
            pltpu.SemaphoreType.DMA,
        ],
        compiler_params=pltpu.CompilerParams(use_tc_tiling_on_sc=False),
    )
    return f(h_split, ee, src3d, dst3d)


def kernel(x, edge_index, edge_attr, batch, node_emb, W_e, b_e, eps,
           W1, b1, bn1_gamma, bn1_beta, W2, b2, bn_gamma, bn_beta):
    src3d = edge_index[0].reshape(NS, NCHUNK, CHUNK)
    dst3d = edge_index[1].reshape(NS, NCHUNK, CHUNK)

    h0row = node_emb[0:1]

    scales = (1.0 + eps).reshape(L, 1, 1)

    msg0 = _edge_mlp(edge_attr, W_e[0], (b_e[0] + node_emb[0]).reshape(1, D), relu=True)
    p = _sc_scatter(msg0, dst3d)
    h = _dense(h0row, p, scales[0], W1[0], b1[0], bn1_gamma[0], bn1_beta[0],
               W2[0], b2[0], bn_gamma[0], bn_beta[0], h_is_row=True, split_out=True)

    for l in range(1, L):
        ee = _edge_mlp(edge_attr, W_e[l], b_e[l].reshape(1, D), relu=False)
        p = _sc_msg(h, ee, src3d, dst3d)
        h = _dense(h, p, scales[l], W1[l], b1[l], bn1_gamma[l], bn1_beta[l],
                   W2[l], b2[l], bn_gamma[l], bn_beta[l],
                   h_is_row=False, split_out=(l != L - 1))
    return h
